# baseline (speedup 1.0000x reference)
import functools

import jax
import jax.numpy as jnp
from jax.experimental import pallas as pl
from jax.experimental.pallas import tpu as pltpu
from jax.experimental.pallas import tpu_sc as plsc

BLK = 128


def _router_body(x_ref, gw_ref, e0, e1, r0, r1, w0r, w1r, cnt, run_ref, *, n_e):
    t = pl.program_id(0)

    @pl.when(t == 0)
    def _():
        run_ref[...] = jnp.zeros_like(run_ref)

    x = x_ref[...]
    bt = x.shape[0]
    logits = jax.lax.dot_general(
        x, gw_ref[...], (((1,), (1,)), ((), ())),
        preferred_element_type=jnp.float32)
    i8 = jax.lax.broadcasted_iota(jnp.int32, (bt, n_e), 1)
    v1 = jnp.max(logits, axis=1, keepdims=True)
    i1 = jnp.min(jnp.where(logits == v1, i8, n_e), axis=1, keepdims=True)
    l2 = jnp.where(i8 == i1, -jnp.inf, logits)
    v2 = jnp.max(l2, axis=1, keepdims=True)
    i2 = jnp.min(jnp.where(l2 == v2, i8, n_e), axis=1, keepdims=True)
    ex = jnp.exp(v2 - v1)
    p1 = 1.0 / (1.0 + ex)
    p2 = ex / (1.0 + ex)
    oh = (i8 == i1).astype(jnp.float32) + (i8 == i2).astype(jnp.float32)
    rowi = jax.lax.broadcasted_iota(jnp.int32, (bt, bt), 0)
    colj = jax.lax.broadcasted_iota(jnp.int32, (bt, bt), 1)
    tril = (rowi > colj).astype(jnp.float32)
    run = jax.lax.dot_general(
        tril, oh, (((1,), (0,)), ((), ())),
        preferred_element_type=jnp.float32)
    tot = run + run_ref[...]
    r0v = jnp.sum(jnp.where(i8 == i1, tot, 0.0), axis=1, keepdims=True)
    r1v = jnp.sum(jnp.where(i8 == i2, tot, 0.0), axis=1, keepdims=True)
    e0[...] = i1
    e1[...] = i2
    r0[...] = r0v.astype(jnp.int32)
    r1[...] = r1v.astype(jnp.int32)
    w0r[...] = p1
    w1r[...] = p2
    run_ref[...] += jnp.sum(oh, axis=0, keepdims=True)

    @pl.when(t == pl.num_programs(0) - 1)
    def _():
        cnt[...] = run_ref[...].astype(jnp.int32)


def _gmm_body(be_ref, xs_ref, ew_ref, eb_ref, ws_ref, ys_ref):
    x = xs_ref[...].astype(jnp.bfloat16)
    y = jax.lax.dot_general(
        x, ew_ref[0], (((1,), (1,)), ((), ())),
        preferred_element_type=jnp.float32) + eb_ref[0]
    ys_ref[...] = ws_ref[...] * y


def _destgen_body(e0_ref, e1_ref, r0_ref, r1_ref, offs_ref, d0_ref, d1_ref):
    n = e0_ref.shape[0]
    i16 = jax.lax.broadcasted_iota(jnp.int32, (n, 16), 1)
    offs_row = offs_ref[...]
    o0 = jnp.sum(jnp.where(i16 == e0_ref[...], offs_row, 0),
                 axis=1, keepdims=True)
    o1 = jnp.sum(jnp.where(i16 == e1_ref[...], offs_row, 0),
                 axis=1, keepdims=True)
    d0_ref[...] = r0_ref[...] + o0
    d1_ref[...] = r1_ref[...] + o1


def _dispatch_body(x_hbm, d0_hbm, d1_hbm, w0_hbm, w1_hbm,
                   xs_hbm, ws_hbm,
                   d0a, d1a, w0a, w1a, x_a, x_b, sem_a, sem_b, sem_s,
                   *, tpw, ch):
    wid = jax.lax.axis_index("s") * 2 + jax.lax.axis_index("c")
    base = wid * tpw
    nch = tpw // ch
    pltpu.sync_copy(d0_hbm.at[pl.ds(base, tpw)], d0a)
    pltpu.sync_copy(d1_hbm.at[pl.ds(base, tpw)], d1a)
    pltpu.sync_copy(w0_hbm.at[pl.ds(base, tpw)], w0a)
    pltpu.sync_copy(w1_hbm.at[pl.ds(base, tpw)], w1a)
    pltpu.async_copy(x_hbm.at[pl.ds(base, ch)], x_a, sem_a)

    def scatter_chunk(c, xbuf):
        off = c * ch
        d0 = d0a[pl.ds(off, ch)]
        d1 = d1a[pl.ds(off, ch)]
        hs = (pltpu.async_copy(xbuf, xs_hbm.at[d0], sem_s),
              pltpu.async_copy(xbuf, xs_hbm.at[d1], sem_s),
              pltpu.async_copy(w0a.at[pl.ds(off, ch)], ws_hbm.at[d0], sem_s),
              pltpu.async_copy(w1a.at[pl.ds(off, ch)], ws_hbm.at[d1], sem_s))
        for h in hs:
            h.wait()

    def body(i, carry):
        ca = 2 * i
        cb = 2 * i + 1
        pltpu.async_copy(x_hbm.at[pl.ds(base + cb * ch, ch)], x_b, sem_b)
        pltpu.make_async_copy(x_hbm.at[pl.ds(0, ch)], x_a, sem_a).wait()
        scatter_chunk(ca, x_a)

        @pl.when(cb + 1 < nch)
        def _():
            pltpu.async_copy(
                x_hbm.at[pl.ds(base + (cb + 1) * ch, ch)], x_a, sem_a)

        pltpu.make_async_copy(x_hbm.at[pl.ds(0, ch)], x_b, sem_b).wait()
        scatter_chunk(cb, x_b)
        return carry

    jax.lax.fori_loop(0, nch // 2, body, 0)


def _combine_body(ys_hbm, d0_hbm, d1_hbm, out_hbm,
                  d0a, d1a, a_a, b_a, a_b, b_b,
                  sem_a, sem_b, sem_oa, sem_ob,
                  *, tpw, ch, d_out):
    wid = jax.lax.axis_index("s") * 2 + jax.lax.axis_index("c")
    base = wid * tpw
    nch = tpw // ch
    pltpu.sync_copy(d0_hbm.at[pl.ds(base, tpw)], d0a)
    pltpu.sync_copy(d1_hbm.at[pl.ds(base, tpw)], d1a)

    def start_gather(c, abuf, bbuf, sem):
        off = c * ch
        pltpu.async_copy(ys_hbm.at[d0a.at[pl.ds(off, ch)]], abuf, sem)
        pltpu.async_copy(ys_hbm.at[d1a.at[pl.ds(off, ch)]], bbuf, sem)

    def drain_gather(abuf, bbuf, sem):
        pltpu.make_async_copy(ys_hbm.at[pl.ds(0, ch)], abuf, sem).wait()
        pltpu.make_async_copy(ys_hbm.at[pl.ds(0, ch)], bbuf, sem).wait()

    def drain_store(abuf, sem):
        pltpu.make_async_copy(out_hbm.at[pl.ds(0, ch)], abuf, sem).wait()

    def add_into(abuf, bbuf):
        def add_k(k, c2):
            for j in range(ch):
                abuf[j, pl.ds(k * 16, 16)] += bbuf[j, pl.ds(k * 16, 16)]
            return c2
        jax.lax.fori_loop(0, d_out // 16, add_k, 0)

    start_gather(0, a_a, b_a, sem_a)

    def body(i, carry):
        ca = 2 * i
        cb = 2 * i + 1

        @pl.when(i > 0)
        def _():
            drain_store(a_b, sem_ob)

        start_gather(cb, a_b, b_b, sem_b)
        drain_gather(a_a, b_a, sem_a)
        add_into(a_a, b_a)
        pltpu.async_copy(a_a, out_hbm.at[pl.ds(base + ca * ch, ch)], sem_oa)

        @pl.when(cb + 1 < nch)
        def _():
            drain_store(a_a, sem_oa)
            start_gather(cb + 1, a_a, b_a, sem_a)

        drain_gather(a_b, b_b, sem_b)
        add_into(a_b, b_b)
        pltpu.async_copy(a_b, out_hbm.at[pl.ds(base + cb * ch, ch)], sem_ob)
        return carry

    jax.lax.fori_loop(0, nch // 2, body, 0)
    drain_store(a_a, sem_oa)
    drain_store(a_b, sem_ob)


def kernel(inputs, gate_w, expert_w, expert_b):
    batch_shape = inputs.shape[:-1]
    d_in = inputs.shape[-1]
    x = inputs.reshape(-1, d_in)
    d_out = expert_b.shape[1]
    ew16 = expert_w.astype(jnp.bfloat16)
    eb3 = expert_b.reshape(expert_b.shape[0], 1, d_out)
    out = _pipeline(x, gate_w, ew16, eb3)
    return out.reshape(*batch_shape, d_out)


def _pipeline(x, gate_w, ew16, eb3):
    d_in = x.shape[1]
    t_tot = x.shape[0]
    n_e = eb3.shape[0]
    d_out = eb3.shape[2]
    n_pairs = 2 * t_tot
    n_blocks = n_pairs // BLK + n_e
    p_tot = n_blocks * BLK
    bt = 512
    n_workers = 32
    tpw = t_tot // n_workers
    ch = 16
    cch = 8

    i32 = jnp.int32
    f32 = jnp.float32
    outs = pl.pallas_call(
        functools.partial(_router_body, n_e=n_e),
        grid=(t_tot // bt,),
        in_specs=[
            pl.BlockSpec((bt, d_in), lambda t: (t, 0)),
            pl.BlockSpec((n_e, d_in), lambda t: (0, 0)),
        ],
        out_specs=[
            pl.BlockSpec((bt, 1), lambda t: (t, 0)),
            pl.BlockSpec((bt, 1), lambda t: (t, 0)),
            pl.BlockSpec((bt, 1), lambda t: (t, 0)),
            pl.BlockSpec((bt, 1), lambda t: (t, 0)),
            pl.BlockSpec((bt, 1), lambda t: (t, 0)),
            pl.BlockSpec((bt, 1), lambda t: (t, 0)),
            pl.BlockSpec((1, n_e), lambda t: (0, 0)),
        ],
        out_shape=[
            jax.ShapeDtypeStruct((t_tot, 1), i32),
            jax.ShapeDtypeStruct((t_tot, 1), i32),
            jax.ShapeDtypeStruct((t_tot, 1), i32),
            jax.ShapeDtypeStruct((t_tot, 1), i32),
            jax.ShapeDtypeStruct((t_tot, 1), f32),
            jax.ShapeDtypeStruct((t_tot, 1), f32),
            jax.ShapeDtypeStruct((1, n_e), i32),
        ],
        scratch_shapes=[pltpu.VMEM((1, n_e), f32)],
        compiler_params=pltpu.CompilerParams(
            dimension_semantics=("arbitrary",)),
    )(x, gate_w)
    e0, e1, r0, r1, w0r, w1r, counts = outs

    c = counts[0]
    padded = ((c + BLK - 1) // BLK) * BLK
    offs = jnp.concatenate([jnp.zeros((1,), i32),
                            jnp.cumsum(padded).astype(i32)])[:n_e]
    offs16 = jnp.pad(offs, (0, 16 - n_e))
    ends = (offs + padded) // BLK
    bid = jnp.arange(n_blocks, dtype=i32)
    be = jnp.minimum(
        jnp.sum((bid[:, None] >= ends[None, :]).astype(i32), axis=1),
        n_e - 1).astype(i32)

    d0a, d1a = pl.pallas_call(
        _destgen_body,
        grid=(1,),
        in_specs=[pl.BlockSpec((t_tot, 1), lambda i: (0, 0))] * 4
        + [pl.BlockSpec((1, 16), lambda i: (0, 0))],
        out_specs=[pl.BlockSpec((t_tot, 1), lambda i: (0, 0))] * 2,
        out_shape=[jax.ShapeDtypeStruct((t_tot, 1), i32)] * 2,
    )(e0, e1, r0, r1, offs16.reshape(1, 16))
    d0f = d0a.reshape(t_tot)
    d1f = d1a.reshape(t_tot)

    mesh = plsc.VectorSubcoreMesh(core_axis_name="c", subcore_axis_name="s")

    disp = pl.kernel(
        functools.partial(_dispatch_body, tpw=tpw, ch=ch),
        out_type=(jax.ShapeDtypeStruct((p_tot, d_in), f32),
                  jax.ShapeDtypeStruct((p_tot,), f32)),
        mesh=mesh,
        scratch_types=[
            pltpu.VMEM((tpw,), i32),
            pltpu.VMEM((tpw,), i32),
            pltpu.VMEM((tpw,), f32),
            pltpu.VMEM((tpw,), f32),
            pltpu.VMEM((ch, d_in), f32),
            pltpu.VMEM((ch, d_in), f32),
            pltpu.SemaphoreType.DMA,
            pltpu.SemaphoreType.DMA,
            pltpu.SemaphoreType.DMA,
        ],
    )
    xs, ws = disp(x, d0f, d1f, w0r.reshape(t_tot), w1r.reshape(t_tot))

    ys = pl.pallas_call(
        _gmm_body,
        grid_spec=pltpu.PrefetchScalarGridSpec(
            num_scalar_prefetch=1,
            grid=(n_blocks,),
            in_specs=[
                pl.BlockSpec((BLK, d_in), lambda b, be_r: (b, 0)),
                pl.BlockSpec((1, d_out, d_in), lambda b, be_r: (be_r[b], 0, 0)),
                pl.BlockSpec((1, 1, d_out), lambda b, be_r: (be_r[b], 0, 0)),
                pl.BlockSpec((BLK, 1), lambda b, be_r: (b, 0)),
            ],
            out_specs=pl.BlockSpec((BLK, d_out), lambda b, be_r: (b, 0)),
        ),
        out_shape=jax.ShapeDtypeStruct((p_tot, d_out), f32),
        compiler_params=pltpu.CompilerParams(
            dimension_semantics=("arbitrary",)),
    )(be, xs, ew16, eb3, ws.reshape(p_tot, 1))

    comb = pl.kernel(
        functools.partial(_combine_body, tpw=tpw, ch=cch, d_out=d_out),
        out_type=jax.ShapeDtypeStruct((t_tot, d_out), f32),
        mesh=mesh,
        scratch_types=[
            pltpu.VMEM((tpw,), i32),
            pltpu.VMEM((tpw,), i32),
            pltpu.VMEM((cch, d_out), f32),
            pltpu.VMEM((cch, d_out), f32),
            pltpu.VMEM((cch, d_out), f32),
            pltpu.VMEM((cch, d_out), f32),
            pltpu.SemaphoreType.DMA,
            pltpu.SemaphoreType.DMA,
            pltpu.SemaphoreType.DMA,
            pltpu.SemaphoreType.DMA,
        ],
    )
    return comb(ys, d0f, d1f)


# speedup vs baseline: 1.5364x; 1.5364x over previous
import functools

import jax
import jax.numpy as jnp
from jax.experimental import pallas as pl
from jax.experimental.pallas import tpu as pltpu
from jax.experimental.pallas import tpu_sc as plsc

BLK = 256


def _router_body(x_ref, gw_ref, e0, e1, r0, r1, w0r, w1r, cnt, run_ref, *, n_e):
    t = pl.program_id(0)

    @pl.when(t == 0)
    def _():
        run_ref[...] = jnp.zeros_like(run_ref)

    x = x_ref[...]
    bt = x.shape[0]
    logits = jax.lax.dot_general(
        x, gw_ref[...], (((1,), (1,)), ((), ())),
        preferred_element_type=jnp.float32)
    i8 = jax.lax.broadcasted_iota(jnp.int32, (bt, n_e), 1)
    v1 = jnp.max(logits, axis=1, keepdims=True)
    i1 = jnp.min(jnp.where(logits == v1, i8, n_e), axis=1, keepdims=True)
    l2 = jnp.where(i8 == i1, -jnp.inf, logits)
    v2 = jnp.max(l2, axis=1, keepdims=True)
    i2 = jnp.min(jnp.where(l2 == v2, i8, n_e), axis=1, keepdims=True)
    ex = jnp.exp(v2 - v1)
    p1 = 1.0 / (1.0 + ex)
    p2 = ex / (1.0 + ex)
    oh = (i8 == i1).astype(jnp.float32) + (i8 == i2).astype(jnp.float32)
    rowi = jax.lax.broadcasted_iota(jnp.int32, (bt, bt), 0)
    colj = jax.lax.broadcasted_iota(jnp.int32, (bt, bt), 1)
    tril = (rowi > colj).astype(jnp.float32)
    run = jax.lax.dot_general(
        tril, oh, (((1,), (0,)), ((), ())),
        preferred_element_type=jnp.float32)
    tot = run + run_ref[...]
    r0v = jnp.sum(jnp.where(i8 == i1, tot, 0.0), axis=1, keepdims=True)
    r1v = jnp.sum(jnp.where(i8 == i2, tot, 0.0), axis=1, keepdims=True)
    e0[...] = i1
    e1[...] = i2
    r0[...] = r0v.astype(jnp.int32)
    r1[...] = r1v.astype(jnp.int32)
    w0r[...] = p1
    w1r[...] = p2
    run_ref[...] += jnp.sum(oh, axis=0, keepdims=True)

    @pl.when(t == pl.num_programs(0) - 1)
    def _():
        cnt[...] = run_ref[...].astype(jnp.int32)


def _gmm_body(be_ref, xs_ref, ew_ref, eb_ref, ws_ref, ys_ref):
    x = xs_ref[...]
    y = jax.lax.dot_general(
        x, ew_ref[0], (((1,), (1,)), ((), ())),
        preferred_element_type=jnp.float32) + eb_ref[0]
    ys_ref[...] = ws_ref[...] * y


def _destgen_body(e0_ref, e1_ref, r0_ref, r1_ref, offs_ref, d0_ref, d1_ref):
    n = e0_ref.shape[0]
    i16 = jax.lax.broadcasted_iota(jnp.int32, (n, 16), 1)
    offs_row = offs_ref[...]
    o0 = jnp.sum(jnp.where(i16 == e0_ref[...], offs_row, 0),
                 axis=1, keepdims=True)
    o1 = jnp.sum(jnp.where(i16 == e1_ref[...], offs_row, 0),
                 axis=1, keepdims=True)
    d0_ref[...] = r0_ref[...] + o0
    d1_ref[...] = r1_ref[...] + o1


def _dispatch_body(x_hbm, d0_hbm, d1_hbm, w0_hbm, w1_hbm,
                   xs_hbm, ws_hbm,
                   d0a, d1a, w0a, w1a, x_a, x_b, sem_a, sem_b, sem_s,
                   *, tpw, ch):
    wid = jax.lax.axis_index("s") * 2 + jax.lax.axis_index("c")
    base = wid * tpw
    nch = tpw // ch
    pltpu.sync_copy(d0_hbm.at[pl.ds(base, tpw)], d0a)
    pltpu.sync_copy(d1_hbm.at[pl.ds(base, tpw)], d1a)
    pltpu.sync_copy(w0_hbm.at[pl.ds(base, tpw)], w0a)
    pltpu.sync_copy(w1_hbm.at[pl.ds(base, tpw)], w1a)
    pltpu.async_copy(x_hbm.at[pl.ds(base, ch)], x_a, sem_a)

    def scatter_chunk(c, xbuf):
        off = c * ch
        d0 = d0a[pl.ds(off, ch)]
        d1 = d1a[pl.ds(off, ch)]
        hs = (pltpu.async_copy(xbuf, xs_hbm.at[d0], sem_s),
              pltpu.async_copy(xbuf, xs_hbm.at[d1], sem_s),
              pltpu.async_copy(w0a.at[pl.ds(off, ch)], ws_hbm.at[d0], sem_s),
              pltpu.async_copy(w1a.at[pl.ds(off, ch)], ws_hbm.at[d1], sem_s))
        for h in hs:
            h.wait()

    def body(i, carry):
        ca = 2 * i
        cb = 2 * i + 1
        pltpu.async_copy(x_hbm.at[pl.ds(base + cb * ch, ch)], x_b, sem_b)
        pltpu.make_async_copy(x_hbm.at[pl.ds(0, ch)], x_a, sem_a).wait()
        scatter_chunk(ca, x_a)

        @pl.when(cb + 1 < nch)
        def _():
            pltpu.async_copy(
                x_hbm.at[pl.ds(base + (cb + 1) * ch, ch)], x_a, sem_a)

        pltpu.make_async_copy(x_hbm.at[pl.ds(0, ch)], x_b, sem_b).wait()
        scatter_chunk(cb, x_b)
        return carry

    jax.lax.fori_loop(0, nch // 2, body, 0)


def _combine_body(ys_hbm, d0_hbm, d1_hbm, out_hbm,
                  d0a, d1a, a_a, b_a, a_b, b_b,
                  sem_a, sem_b, sem_oa, sem_ob,
                  *, tpw, ch, d_out):
    wid = jax.lax.axis_index("s") * 2 + jax.lax.axis_index("c")
    base = wid * tpw
    nch = tpw // ch
    pltpu.sync_copy(d0_hbm.at[pl.ds(base, tpw)], d0a)
    pltpu.sync_copy(d1_hbm.at[pl.ds(base, tpw)], d1a)

    def start_gather(c, abuf, bbuf, sem):
        off = c * ch
        pltpu.async_copy(ys_hbm.at[d0a.at[pl.ds(off, ch)]], abuf, sem)
        pltpu.async_copy(ys_hbm.at[d1a.at[pl.ds(off, ch)]], bbuf, sem)

    def drain_gather(abuf, bbuf, sem):
        pltpu.make_async_copy(ys_hbm.at[pl.ds(0, ch)], abuf, sem).wait()
        pltpu.make_async_copy(ys_hbm.at[pl.ds(0, ch)], bbuf, sem).wait()

    def drain_store(abuf, sem):
        pltpu.make_async_copy(out_hbm.at[pl.ds(0, ch)], abuf, sem).wait()

    def add_into(abuf, bbuf):
        def add_k(k, c2):
            for j in range(ch):
                abuf[j, pl.ds(k * 16, 16)] += bbuf[j, pl.ds(k * 16, 16)]
            return c2
        jax.lax.fori_loop(0, d_out // 16, add_k, 0)

    start_gather(0, a_a, b_a, sem_a)

    def body(i, carry):
        ca = 2 * i
        cb = 2 * i + 1

        @pl.when(i > 0)
        def _():
            drain_store(a_b, sem_ob)

        start_gather(cb, a_b, b_b, sem_b)
        drain_gather(a_a, b_a, sem_a)
        add_into(a_a, b_a)
        pltpu.async_copy(a_a, out_hbm.at[pl.ds(base + ca * ch, ch)], sem_oa)

        @pl.when(cb + 1 < nch)
        def _():
            drain_store(a_a, sem_oa)
            start_gather(cb + 1, a_a, b_a, sem_a)

        drain_gather(a_b, b_b, sem_b)
        add_into(a_b, b_b)
        pltpu.async_copy(a_b, out_hbm.at[pl.ds(base + cb * ch, ch)], sem_ob)
        return carry

    jax.lax.fori_loop(0, nch // 2, body, 0)
    drain_store(a_a, sem_oa)
    drain_store(a_b, sem_ob)


def kernel(inputs, gate_w, expert_w, expert_b):
    batch_shape = inputs.shape[:-1]
    d_in = inputs.shape[-1]
    x = inputs.reshape(-1, d_in)
    d_out = expert_b.shape[1]
    ew16 = expert_w
    eb3 = expert_b.reshape(expert_b.shape[0], 1, d_out)
    out = _pipeline(x, gate_w, ew16, eb3)
    return out.reshape(*batch_shape, d_out)


def _pipeline(x, gate_w, ew16, eb3):
    d_in = x.shape[1]
    t_tot = x.shape[0]
    n_e = eb3.shape[0]
    d_out = eb3.shape[2]
    n_pairs = 2 * t_tot
    n_blocks = n_pairs // BLK + n_e
    p_tot = n_blocks * BLK
    bt = 512
    n_workers = 32
    tpw = t_tot // n_workers
    ch = 16
    cch = 8

    i32 = jnp.int32
    f32 = jnp.float32
    outs = pl.pallas_call(
        functools.partial(_router_body, n_e=n_e),
        grid=(t_tot // bt,),
        in_specs=[
            pl.BlockSpec((bt, d_in), lambda t: (t, 0)),
            pl.BlockSpec((n_e, d_in), lambda t: (0, 0)),
        ],
        out_specs=[
            pl.BlockSpec((bt, 1), lambda t: (t, 0)),
            pl.BlockSpec((bt, 1), lambda t: (t, 0)),
            pl.BlockSpec((bt, 1), lambda t: (t, 0)),
            pl.BlockSpec((bt, 1), lambda t: (t, 0)),
            pl.BlockSpec((bt, 1), lambda t: (t, 0)),
            pl.BlockSpec((bt, 1), lambda t: (t, 0)),
            pl.BlockSpec((1, n_e), lambda t: (0, 0)),
        ],
        out_shape=[
            jax.ShapeDtypeStruct((t_tot, 1), i32),
            jax.ShapeDtypeStruct((t_tot, 1), i32),
            jax.ShapeDtypeStruct((t_tot, 1), i32),
            jax.ShapeDtypeStruct((t_tot, 1), i32),
            jax.ShapeDtypeStruct((t_tot, 1), f32),
            jax.ShapeDtypeStruct((t_tot, 1), f32),
            jax.ShapeDtypeStruct((1, n_e), i32),
        ],
        scratch_shapes=[pltpu.VMEM((1, n_e), f32)],
        compiler_params=pltpu.CompilerParams(
            dimension_semantics=("arbitrary",)),
    )(x, gate_w)
    e0, e1, r0, r1, w0r, w1r, counts = outs

    c = counts[0]
    padded = ((c + BLK - 1) // BLK) * BLK
    offs = jnp.concatenate([jnp.zeros((1,), i32),
                            jnp.cumsum(padded).astype(i32)])[:n_e]
    offs16 = jnp.pad(offs, (0, 16 - n_e))
    ends = (offs + padded) // BLK
    bid = jnp.arange(n_blocks, dtype=i32)
    be = jnp.minimum(
        jnp.sum((bid[:, None] >= ends[None, :]).astype(i32), axis=1),
        n_e - 1).astype(i32)

    d0a, d1a = pl.pallas_call(
        _destgen_body,
        grid=(1,),
        in_specs=[pl.BlockSpec((t_tot, 1), lambda i: (0, 0))] * 4
        + [pl.BlockSpec((1, 16), lambda i: (0, 0))],
        out_specs=[pl.BlockSpec((t_tot, 1), lambda i: (0, 0))] * 2,
        out_shape=[jax.ShapeDtypeStruct((t_tot, 1), i32)] * 2,
    )(e0, e1, r0, r1, offs16.reshape(1, 16))
    d0f = d0a.reshape(t_tot)
    d1f = d1a.reshape(t_tot)

    mesh = plsc.VectorSubcoreMesh(core_axis_name="c", subcore_axis_name="s")

    disp = pl.kernel(
        functools.partial(_dispatch_body, tpw=tpw, ch=ch),
        out_type=(jax.ShapeDtypeStruct((p_tot, d_in), f32),
                  jax.ShapeDtypeStruct((p_tot,), f32)),
        mesh=mesh,
        scratch_types=[
            pltpu.VMEM((tpw,), i32),
            pltpu.VMEM((tpw,), i32),
            pltpu.VMEM((tpw,), f32),
            pltpu.VMEM((tpw,), f32),
            pltpu.VMEM((ch, d_in), f32),
            pltpu.VMEM((ch, d_in), f32),
            pltpu.SemaphoreType.DMA,
            pltpu.SemaphoreType.DMA,
            pltpu.SemaphoreType.DMA,
        ],
    )
    xs, ws = disp(x, d0f, d1f, w0r.reshape(t_tot), w1r.reshape(t_tot))

    ys = pl.pallas_call(
        _gmm_body,
        grid_spec=pltpu.PrefetchScalarGridSpec(
            num_scalar_prefetch=1,
            grid=(n_blocks,),
            in_specs=[
                pl.BlockSpec((BLK, d_in), lambda b, be_r: (b, 0)),
                pl.BlockSpec((1, d_out, d_in), lambda b, be_r: (be_r[b], 0, 0)),
                pl.BlockSpec((1, 1, d_out), lambda b, be_r: (be_r[b], 0, 0)),
                pl.BlockSpec((BLK, 1), lambda b, be_r: (b, 0)),
            ],
            out_specs=pl.BlockSpec((BLK, d_out), lambda b, be_r: (b, 0)),
        ),
        out_shape=jax.ShapeDtypeStruct((p_tot, d_out), f32),
        compiler_params=pltpu.CompilerParams(
            dimension_semantics=("arbitrary",)),
    )(be, xs, ew16, eb3, ws.reshape(p_tot, 1))

    comb = pl.kernel(
        functools.partial(_combine_body, tpw=tpw, ch=cch, d_out=d_out),
        out_type=jax.ShapeDtypeStruct((t_tot, d_out), f32),
        mesh=mesh,
        scratch_types=[
            pltpu.VMEM((tpw,), i32),
            pltpu.VMEM((tpw,), i32),
            pltpu.VMEM((cch, d_out), f32),
            pltpu.VMEM((cch, d_out), f32),
            pltpu.VMEM((cch, d_out), f32),
            pltpu.VMEM((cch, d_out), f32),
            pltpu.SemaphoreType.DMA,
            pltpu.SemaphoreType.DMA,
            pltpu.SemaphoreType.DMA,
            pltpu.SemaphoreType.DMA,
        ],
    )
    return comb(ys, d0f, d1f)
